# baseline (speedup 1.0000x reference)
import functools

import jax
import jax.numpy as jnp
from jax import lax
from jax.experimental import pallas as pl
from jax.experimental.pallas import tpu as pltpu
from jax.experimental.pallas import tpu_sc as plsc

D_MODEL = 768
D_FF = 3072
N_EMB = 16384
K = 64
N_BUCKET = N_EMB // K
TOK_CHUNK = 8
BUCKETS_PER_STEP = 8

_INTERPRET = False


def _maxabs_kernel(x_ref, o_ref):
    @pl.when(pl.program_id(0) == 0)
    def _init():
        o_ref[...] = jnp.zeros_like(o_ref)
    o_ref[...] = jnp.maximum(o_ref[...],
                             jnp.max(jnp.abs(x_ref[...])).reshape(1, 1))


def _global_maxabs(x, tile_rows):
    rows = x.shape[0]
    grid = rows // tile_rows
    return pl.pallas_call(
        _maxabs_kernel,
        grid=(grid,),
        in_specs=[pl.BlockSpec((tile_rows, x.shape[1]), lambda i: (i, 0))],
        out_specs=pl.BlockSpec((1, 1), lambda i: (0, 0)),
        out_shape=jax.ShapeDtypeStruct((1, 1), jnp.float32),
        interpret=_INTERPRET,
    )(x)


def _maxabs2_kernel(a_ref, b_ref, oa_ref, ob_ref):
    @pl.when(pl.program_id(0) == 0)
    def _init():
        oa_ref[...] = jnp.zeros_like(oa_ref)
        ob_ref[...] = jnp.zeros_like(ob_ref)
    oa_ref[...] = jnp.maximum(oa_ref[...],
                              jnp.max(jnp.abs(a_ref[...])).reshape(1, 1))
    ob_ref[...] = jnp.maximum(ob_ref[...],
                              jnp.max(jnp.abs(b_ref[...])).reshape(1, 1))


def _global_maxabs2(a, b, tile_cols):
    grid = a.shape[1] // tile_cols
    return pl.pallas_call(
        _maxabs2_kernel,
        grid=(grid,),
        in_specs=[
            pl.BlockSpec((a.shape[0], tile_cols), lambda i: (0, i)),
            pl.BlockSpec((a.shape[0], tile_cols), lambda i: (0, i)),
        ],
        out_specs=[
            pl.BlockSpec((1, 1), lambda i: (0, 0)),
            pl.BlockSpec((1, 1), lambda i: (0, 0)),
        ],
        out_shape=[
            jax.ShapeDtypeStruct((1, 1), jnp.float32),
            jax.ShapeDtypeStruct((1, 1), jnp.float32),
        ],
        interpret=_INTERPRET,
    )(a, b)


def _xsim_ln_kernel(x_ref, xmax_ref, g_ref, b_ref, xsim_ref, a_ref):
    s = 32767.0 / jnp.maximum(xmax_ref[...], 1e-8)
    xq = jnp.round(x_ref[...] * s) / s
    xsim_ref[...] = xq
    m = jnp.mean(xq, axis=-1, keepdims=True)
    v = jnp.mean((xq - m) ** 2, axis=-1, keepdims=True)
    a_ref[...] = (xq - m) * lax.rsqrt(v + 1e-5) * g_ref[...] + b_ref[...]


def _quantize_x_ln(x2d, xmax, ln_g, ln_b, tile_rows):
    grid = x2d.shape[0] // tile_rows
    return pl.pallas_call(
        _xsim_ln_kernel,
        grid=(grid,),
        in_specs=[
            pl.BlockSpec((tile_rows, D_MODEL), lambda i: (i, 0)),
            pl.BlockSpec((1, 1), lambda i: (0, 0)),
            pl.BlockSpec((1, D_MODEL), lambda i: (0, 0)),
            pl.BlockSpec((1, D_MODEL), lambda i: (0, 0)),
        ],
        out_specs=[
            pl.BlockSpec((tile_rows, D_MODEL), lambda i: (i, 0)),
            pl.BlockSpec((tile_rows, D_MODEL), lambda i: (i, 0)),
        ],
        out_shape=[
            jax.ShapeDtypeStruct(x2d.shape, jnp.float32),
            jax.ShapeDtypeStruct(x2d.shape, jnp.float32),
        ],
        interpret=_INTERPRET,
    )(x2d, xmax, ln_g.reshape(1, -1), ln_b.reshape(1, -1))


SC_NUM_CORES = 2
SC_NUM_SUBCORES = 16
SC_WORKERS = SC_NUM_CORES * SC_NUM_SUBCORES


def _sc_gather_rows(src, idx, n_out):
    assert n_out % (8 * SC_WORKERS) == 0
    b_per_w = n_out // SC_WORKERS

    cs = 16
    nch = b_per_w // cs

    def body(src_hbm, idx_hbm, out_hbm, idx_v, rows_v, gsem, wsem):
        wid = lax.axis_index("s") * SC_NUM_CORES + lax.axis_index("c")
        base = wid * b_per_w
        pltpu.sync_copy(idx_hbm.at[pl.ds(base, b_per_w)], idx_v)
        copies = [
            pltpu.async_copy(src_hbm.at[idx_v.at[pl.ds(c * cs, cs)]],
                             rows_v.at[pl.ds(c * cs, cs)], gsem)
            for c in range(nch)
        ]
        wbs = []
        for c, cp in enumerate(copies):
            cp.wait()
            wbs.append(pltpu.async_copy(
                rows_v.at[pl.ds(c * cs, cs)],
                out_hbm.at[pl.ds(base + c * cs, cs)], wsem))
        for wb in wbs:
            wb.wait()

    return pl.kernel(
        body,
        out_type=jax.ShapeDtypeStruct((n_out, D_MODEL), jnp.float32),
        mesh=plsc.VectorSubcoreMesh(core_axis_name="c", subcore_axis_name="s"),
        scratch_types=[
            pltpu.VMEM((b_per_w,), jnp.int32),
            pltpu.VMEM((b_per_w, D_MODEL), jnp.float32),
            pltpu.SemaphoreType.DMA,
            pltpu.SemaphoreType.DMA,
        ],
        interpret=_INTERPRET,
    )(src, idx)


TC_G = 128


def _tc_gather_kernel(idx_ref, src_ref, o_ref):
    i = pl.program_id(0)
    for j in range(TC_G):
        o_ref[j, :] = src_ref[idx_ref[i * TC_G + j], :]


def _tc_gather_rows(src, idx, n_out):
    return pl.pallas_call(
        _tc_gather_kernel,
        grid_spec=pltpu.PrefetchScalarGridSpec(
            num_scalar_prefetch=1,
            grid=(n_out // TC_G,),
            in_specs=[pl.BlockSpec(src.shape, lambda i, idx: (0, 0))],
            out_specs=pl.BlockSpec((TC_G, D_MODEL), lambda i, idx: (i, 0)),
        ),
        out_shape=jax.ShapeDtypeStruct((n_out, D_MODEL), jnp.float32),
        interpret=_INTERPRET,
    )(idx, src)


GATHER_ROWS = 32


def _gather_ln_kernel(perm_ref, *refs):
    row_refs = refs[:GATHER_ROWS]
    g_ref, b_ref, o_ref = refs[GATHER_ROWS], refs[GATHER_ROWS + 1], refs[-1]
    rows = jnp.concatenate([r[...].reshape(1, D_MODEL) for r in row_refs],
                           axis=0)
    m = jnp.mean(rows, axis=-1, keepdims=True)
    v = jnp.mean((rows - m) ** 2, axis=-1, keepdims=True)
    o_ref[...] = (rows - m) * lax.rsqrt(v + 1e-5) * g_ref[...] + b_ref[...]


def _gather_ln(x_sim, perm_pad, ln_g, ln_b, n_pad):
    G = GATHER_ROWS
    grid = n_pad // G
    x3d = x_sim.reshape(-1, 1, D_MODEL)
    row_specs = [
        pl.BlockSpec((1, 1, D_MODEL),
                     functools.partial(
                         lambda i, perm, j: (perm[i * G + j], 0, 0), j=j))
        for j in range(G)
    ]
    return pl.pallas_call(
        _gather_ln_kernel,
        grid_spec=pltpu.PrefetchScalarGridSpec(
            num_scalar_prefetch=1,
            grid=(grid,),
            in_specs=row_specs + [
                pl.BlockSpec((1, D_MODEL), lambda i, perm: (0, 0)),
                pl.BlockSpec((1, D_MODEL), lambda i, perm: (0, 0)),
            ],
            out_specs=pl.BlockSpec((G, D_MODEL), lambda i, perm: (i, 0)),
        ),
        out_shape=jax.ShapeDtypeStruct((n_pad, D_MODEL), jnp.float32),
        interpret=_INTERPRET,
    )(perm_pad, *([x3d] * G), ln_g.reshape(1, -1), ln_b.reshape(1, -1))


def _gather_kernel(inv_ref, *refs):
    row_refs = refs[:GATHER_ROWS]
    o_ref = refs[-1]
    o_ref[...] = jnp.concatenate(
        [r[...].reshape(1, D_MODEL) for r in row_refs], axis=0)


def _gather_rows(src, inv_perm, n_out):
    G = GATHER_ROWS
    grid = n_out // G
    src3d = src.reshape(-1, 1, D_MODEL)
    row_specs = [
        pl.BlockSpec((1, 1, D_MODEL),
                     functools.partial(
                         lambda i, inv, j: (inv[i * G + j], 0, 0), j=j))
        for j in range(G)
    ]
    return pl.pallas_call(
        _gather_kernel,
        grid_spec=pltpu.PrefetchScalarGridSpec(
            num_scalar_prefetch=1,
            grid=(grid,),
            in_specs=row_specs,
            out_specs=pl.BlockSpec((G, D_MODEL), lambda i, inv: (i, 0)),
        ),
        out_shape=jax.ShapeDtypeStruct((n_out, D_MODEL), jnp.float32),
        interpret=_INTERPRET,
    )(inv_perm, *([src3d] * G))


def _attn_kernel(off8_ref, cnt_ref, wk_ref, wv_ref, kmax_ref, vmax_ref,
                 a_ref, o_ref):
    step = pl.program_id(0)
    ks = 127.0 / jnp.maximum(kmax_ref[...], 1e-8)
    vs = 127.0 / jnp.maximum(vmax_ref[...], 1e-8)
    kq_all = jnp.round(wk_ref[...] * ks)
    vq_all = jnp.round(wv_ref[...] * vs)
    inv_ks = 1.0 / ks
    inv_vs = 1.0 / vs

    for j in range(BUCKETS_PER_STEP):
        b = step * BUCKETS_PER_STEP + j
        start8 = off8_ref[b]
        cnt = cnt_ref[b]
        end = start8 * 8 + cnt
        nchunks = (cnt + TOK_CHUNK - 1) // TOK_CHUNK
        kq = kq_all[:, j * K:(j + 1) * K]
        vq = vq_all[:, j * K:(j + 1) * K]

        def chunk_body(c, _, start8=start8, end=end, kq=kq, vq=vq):
            s0 = (start8 + c * (TOK_CHUNK // 8)) * 8
            a_chunk = a_ref[pl.ds(s0, TOK_CHUNK), :]
            s = jnp.dot(a_chunk, kq, preferred_element_type=jnp.float32)
            s = s * inv_ks
            s = s - jnp.max(s, axis=-1, keepdims=True)
            e = jnp.exp(s)
            p = (e * inv_vs) / jnp.sum(e, axis=-1, keepdims=True)
            comb = lax.dot_general(p, vq, (((1,), (1,)), ((), ())),
                                   preferred_element_type=jnp.float32)
            rows = s0 + lax.broadcasted_iota(jnp.int32, (TOK_CHUNK, 1), 0)
            mask = rows < end
            cur = o_ref[pl.ds(s0, TOK_CHUNK), :]
            o_ref[pl.ds(s0, TOK_CHUNK), :] = jnp.where(mask, comb, cur)
            return 0

        lax.fori_loop(0, nchunks, chunk_body, 0)


def _bucket_attention(W_K, W_V, kmax, vmax, a_sorted, off8, cnts, n_pad):
    grid = N_BUCKET // BUCKETS_PER_STEP
    cols = BUCKETS_PER_STEP * K
    return pl.pallas_call(
        _attn_kernel,
        grid_spec=pltpu.PrefetchScalarGridSpec(
            num_scalar_prefetch=2,
            grid=(grid,),
            in_specs=[
                pl.BlockSpec((D_MODEL, cols), lambda i, o, c: (0, i)),
                pl.BlockSpec((D_MODEL, cols), lambda i, o, c: (0, i)),
                pl.BlockSpec((1, 1), lambda i, o, c: (0, 0)),
                pl.BlockSpec((1, 1), lambda i, o, c: (0, 0)),
                pl.BlockSpec((n_pad, D_MODEL), lambda i, o, c: (0, 0)),
            ],
            out_specs=pl.BlockSpec((n_pad, D_MODEL), lambda i, o, c: (0, 0)),
        ),
        out_shape=jax.ShapeDtypeStruct((n_pad, D_MODEL), jnp.float32),
        interpret=_INTERPRET,
    )(off8, cnts, W_K, W_V, kmax, vmax, a_sorted)


def _ffn1_kernel(comb_ref, win_ref, h_ref, hmax_ref):
    cmax = jnp.max(jnp.abs(win_ref[...]), axis=0, keepdims=True)
    cs = 127.0 / jnp.maximum(cmax, 1e-8)
    wq = jnp.round(win_ref[...] * cs) / cs
    h = jnp.dot(comb_ref[...], wq, preferred_element_type=jnp.float32)
    h_ref[...] = h

    @pl.when(pl.program_id(0) == 0)
    def _init():
        hmax_ref[...] = jnp.zeros_like(hmax_ref)
    hmax_ref[...] = jnp.maximum(hmax_ref[...],
                                jnp.max(jnp.abs(h)).reshape(1, 1))


def _ffn1(comb, W_in, tile_rows):
    grid = comb.shape[0] // tile_rows
    return pl.pallas_call(
        _ffn1_kernel,
        grid=(grid,),
        in_specs=[
            pl.BlockSpec((tile_rows, D_MODEL), lambda i: (i, 0)),
            pl.BlockSpec((D_MODEL, D_FF), lambda i: (0, 0)),
        ],
        out_specs=[
            pl.BlockSpec((tile_rows, D_FF), lambda i: (i, 0)),
            pl.BlockSpec((1, 1), lambda i: (0, 0)),
        ],
        out_shape=[
            jax.ShapeDtypeStruct((comb.shape[0], D_FF), jnp.float32),
            jax.ShapeDtypeStruct((1, 1), jnp.float32),
        ],
        interpret=_INTERPRET,
    )(comb, W_in)


def _ffn2_kernel(h_ref, hmax_ref, xsim_ref, wout_ref, wskip_ref,
                 g_ref, b_ref, o_ref):
    hs = 32767.0 / jnp.maximum(hmax_ref[...], 1e-8)
    h = jnp.round(h_ref[...] * hs) / hs
    m = jnp.mean(h, axis=-1, keepdims=True)
    v = jnp.mean((h - m) ** 2, axis=-1, keepdims=True)
    ln = (h - m) * lax.rsqrt(v + 1e-5) * g_ref[...] + b_ref[...]

    rmax = jnp.max(jnp.abs(wout_ref[...]), axis=1, keepdims=True)
    rs = 127.0 / jnp.maximum(rmax, 1e-8)
    woq = jnp.round(wout_ref[...] * rs) / rs

    smax = jnp.max(jnp.abs(wskip_ref[...]))
    ss = 127.0 / jnp.maximum(smax, 1e-8)
    wsq = jnp.round(wskip_ref[...] * ss) / ss

    o_ref[...] = (
        jnp.dot(ln, woq, preferred_element_type=jnp.float32)
        + jnp.dot(xsim_ref[...], wsq, preferred_element_type=jnp.float32)
    )


def _ffn2(hidden, hmax, x_sim, W_out, W_skip, ln2_g, ln2_b, tile_rows):
    grid = hidden.shape[0] // tile_rows
    return pl.pallas_call(
        _ffn2_kernel,
        grid=(grid,),
        in_specs=[
            pl.BlockSpec((tile_rows, D_FF), lambda i: (i, 0)),
            pl.BlockSpec((1, 1), lambda i: (0, 0)),
            pl.BlockSpec((tile_rows, D_MODEL), lambda i: (i, 0)),
            pl.BlockSpec((D_FF, D_MODEL), lambda i: (0, 0)),
            pl.BlockSpec((D_MODEL, D_MODEL), lambda i: (0, 0)),
            pl.BlockSpec((1, D_FF), lambda i: (0, 0)),
            pl.BlockSpec((1, D_FF), lambda i: (0, 0)),
        ],
        out_specs=pl.BlockSpec((tile_rows, D_MODEL), lambda i: (i, 0)),
        out_shape=jax.ShapeDtypeStruct((hidden.shape[0], D_MODEL), jnp.float32),
        interpret=_INTERPRET,
    )(hidden, hmax, x_sim, W_out, W_skip,
      ln2_g.reshape(1, -1), ln2_b.reshape(1, -1))


def kernel(x, input_ids, W_K, W_V, W_in, W_out, W_skip,
           ln1_g, ln1_b, ln2_g, ln2_b):
    B, T, D = x.shape
    n_tok = B * T
    n_pad = 4096
    assert n_tok + N_BUCKET * 7 + TOK_CHUNK <= n_pad
    x2d = x.reshape(n_tok, D)

    ids = input_ids.reshape(n_tok).astype(jnp.int32) % N_BUCKET
    onehot = (ids[:, None] == jnp.arange(N_BUCKET, dtype=jnp.int32)[None, :])
    onehot = onehot.astype(jnp.int32)
    occ = jnp.cumsum(onehot, axis=0)
    counts = occ[-1].astype(jnp.int32)
    rank = jnp.take_along_axis(occ, ids[:, None], axis=1)[:, 0] - 1
    aligned = ((counts + 7) // 8) * 8
    aligned_off = jnp.concatenate([jnp.zeros((1,), jnp.int32),
                                   jnp.cumsum(aligned).astype(jnp.int32)])
    off8 = (aligned_off // 8).astype(jnp.int32)
    inv_pos = (aligned_off[ids] + rank).astype(jnp.int32)
    slot_token = jnp.zeros((n_pad,), jnp.int32).at[inv_pos].set(
        jnp.arange(n_tok, dtype=jnp.int32))

    xmax = _global_maxabs(x2d, 256)
    kmax, vmax = _global_maxabs2(W_K, W_V, 2048)

    x_sim, a_nat = _quantize_x_ln(x2d, xmax, ln1_g, ln1_b, 256)
    a_sorted = _tc_gather_rows(a_nat, slot_token, n_pad)

    comb_sorted = _bucket_attention(W_K, W_V, kmax, vmax, a_sorted,
                                    off8, counts, n_pad)
    comb = _tc_gather_rows(comb_sorted, inv_pos, n_tok)

    hidden, hmax = _ffn1(comb, W_in, 256)
    out = _ffn2(hidden, hmax, x_sim, W_out, W_skip, ln2_g, ln2_b, 256)
    return out.reshape(B, T, D)


# speedup vs baseline: 1.1481x; 1.1481x over previous
import jax
import jax.numpy as jnp
from jax import lax
from jax.experimental import pallas as pl
from jax.experimental.pallas import tpu as pltpu

D_MODEL = 768
D_FF = 3072
N_EMB = 16384
K = 64
N_BUCKET = N_EMB // K
TOK_CHUNK = 16
BUCKETS_PER_STEP = 8

_INTERPRET = False


def _maxabs_kernel(x_ref, o_ref):
    @pl.when(pl.program_id(0) == 0)
    def _init():
        o_ref[...] = jnp.zeros_like(o_ref)
    o_ref[...] = jnp.maximum(o_ref[...],
                             jnp.max(jnp.abs(x_ref[...])).reshape(1, 1))


def _global_maxabs(x, tile_rows):
    rows = x.shape[0]
    grid = rows // tile_rows
    return pl.pallas_call(
        _maxabs_kernel,
        grid=(grid,),
        in_specs=[pl.BlockSpec((tile_rows, x.shape[1]), lambda i: (i, 0))],
        out_specs=pl.BlockSpec((1, 1), lambda i: (0, 0)),
        out_shape=jax.ShapeDtypeStruct((1, 1), jnp.float32),
        interpret=_INTERPRET,
    )(x)


def _maxabs2_kernel(a_ref, b_ref, oa_ref, ob_ref):
    @pl.when(pl.program_id(0) == 0)
    def _init():
        oa_ref[...] = jnp.zeros_like(oa_ref)
        ob_ref[...] = jnp.zeros_like(ob_ref)
    oa_ref[...] = jnp.maximum(oa_ref[...],
                              jnp.max(jnp.abs(a_ref[...])).reshape(1, 1))
    ob_ref[...] = jnp.maximum(ob_ref[...],
                              jnp.max(jnp.abs(b_ref[...])).reshape(1, 1))


def _global_maxabs2(a, b, tile_cols):
    grid = a.shape[1] // tile_cols
    return pl.pallas_call(
        _maxabs2_kernel,
        grid=(grid,),
        in_specs=[
            pl.BlockSpec((a.shape[0], tile_cols), lambda i: (0, i)),
            pl.BlockSpec((a.shape[0], tile_cols), lambda i: (0, i)),
        ],
        out_specs=[
            pl.BlockSpec((1, 1), lambda i: (0, 0)),
            pl.BlockSpec((1, 1), lambda i: (0, 0)),
        ],
        out_shape=[
            jax.ShapeDtypeStruct((1, 1), jnp.float32),
            jax.ShapeDtypeStruct((1, 1), jnp.float32),
        ],
        interpret=_INTERPRET,
    )(a, b)


def _xsim_ln_kernel(x_ref, xmax_ref, g_ref, b_ref, xsim_ref, a_ref):
    s = 32767.0 / jnp.maximum(xmax_ref[...], 1e-8)
    xq = jnp.round(x_ref[...] * s) / s
    xsim_ref[...] = xq
    m = jnp.mean(xq, axis=-1, keepdims=True)
    v = jnp.mean((xq - m) ** 2, axis=-1, keepdims=True)
    a_ref[...] = (xq - m) * lax.rsqrt(v + 1e-5) * g_ref[...] + b_ref[...]


def _quantize_x_ln(x2d, xmax, ln_g, ln_b, tile_rows):
    grid = x2d.shape[0] // tile_rows
    return pl.pallas_call(
        _xsim_ln_kernel,
        grid=(grid,),
        in_specs=[
            pl.BlockSpec((tile_rows, D_MODEL), lambda i: (i, 0)),
            pl.BlockSpec((1, 1), lambda i: (0, 0)),
            pl.BlockSpec((1, D_MODEL), lambda i: (0, 0)),
            pl.BlockSpec((1, D_MODEL), lambda i: (0, 0)),
        ],
        out_specs=[
            pl.BlockSpec((tile_rows, D_MODEL), lambda i: (i, 0)),
            pl.BlockSpec((tile_rows, D_MODEL), lambda i: (i, 0)),
        ],
        out_shape=[
            jax.ShapeDtypeStruct(x2d.shape, jnp.float32),
            jax.ShapeDtypeStruct(x2d.shape, jnp.float32),
        ],
        interpret=_INTERPRET,
    )(x2d, xmax, ln_g.reshape(1, -1), ln_b.reshape(1, -1))


TC_G = 128


def _tc_gather_kernel(idx_ref, src_ref, o_ref):
    i = pl.program_id(0)
    for j in range(TC_G):
        o_ref[j, :] = src_ref[idx_ref[i * TC_G + j], :]


def _tc_gather_rows(src, idx, n_out):
    return pl.pallas_call(
        _tc_gather_kernel,
        grid_spec=pltpu.PrefetchScalarGridSpec(
            num_scalar_prefetch=1,
            grid=(n_out // TC_G,),
            in_specs=[pl.BlockSpec(src.shape, lambda i, idx: (0, 0))],
            out_specs=pl.BlockSpec((TC_G, D_MODEL), lambda i, idx: (i, 0)),
        ),
        out_shape=jax.ShapeDtypeStruct((n_out, D_MODEL), jnp.float32),
        interpret=_INTERPRET,
    )(idx, src)


def _attn_kernel(off8_ref, cnt_ref, wk_ref, wv_ref, kmax_ref, vmax_ref,
                 a_ref, o_ref):
    step = pl.program_id(0)
    ks = 127.0 / jnp.maximum(kmax_ref[...], 1e-8)
    vs = 127.0 / jnp.maximum(vmax_ref[...], 1e-8)
    kq_all = jnp.round(wk_ref[...] * ks)
    vq_all = jnp.round(wv_ref[...] * vs)
    inv_ks = 1.0 / ks
    inv_vs = 1.0 / vs

    for j in range(BUCKETS_PER_STEP):
        b = step * BUCKETS_PER_STEP + j
        start8 = off8_ref[b]
        cnt = cnt_ref[b]
        end = start8 * 8 + cnt
        nchunks = (cnt + TOK_CHUNK - 1) // TOK_CHUNK
        kq = kq_all[:, j * K:(j + 1) * K]
        vq = vq_all[:, j * K:(j + 1) * K]

        def chunk_body(c, _, start8=start8, end=end, kq=kq, vq=vq):
            s0 = (start8 + c * (TOK_CHUNK // 8)) * 8
            a_chunk = a_ref[pl.ds(s0, TOK_CHUNK), :]
            s = jnp.dot(a_chunk, kq, preferred_element_type=jnp.float32)
            s = s * inv_ks
            s = s - jnp.max(s, axis=-1, keepdims=True)
            e = jnp.exp(s)
            p = (e * inv_vs) / jnp.sum(e, axis=-1, keepdims=True)
            comb = lax.dot_general(p, vq, (((1,), (1,)), ((), ())),
                                   preferred_element_type=jnp.float32)
            rows = s0 + lax.broadcasted_iota(jnp.int32, (TOK_CHUNK, 1), 0)
            mask = rows < end
            cur = o_ref[pl.ds(s0, TOK_CHUNK), :]
            o_ref[pl.ds(s0, TOK_CHUNK), :] = jnp.where(mask, comb, cur)
            return 0

        lax.fori_loop(0, nchunks, chunk_body, 0)


def _bucket_attention(W_K, W_V, kmax, vmax, a_sorted, off8, cnts, n_pad):
    grid = N_BUCKET // BUCKETS_PER_STEP
    cols = BUCKETS_PER_STEP * K
    return pl.pallas_call(
        _attn_kernel,
        grid_spec=pltpu.PrefetchScalarGridSpec(
            num_scalar_prefetch=2,
            grid=(grid,),
            in_specs=[
                pl.BlockSpec((D_MODEL, cols), lambda i, o, c: (0, i)),
                pl.BlockSpec((D_MODEL, cols), lambda i, o, c: (0, i)),
                pl.BlockSpec((1, 1), lambda i, o, c: (0, 0)),
                pl.BlockSpec((1, 1), lambda i, o, c: (0, 0)),
                pl.BlockSpec((n_pad, D_MODEL), lambda i, o, c: (0, 0)),
            ],
            out_specs=pl.BlockSpec((n_pad, D_MODEL), lambda i, o, c: (0, 0)),
        ),
        out_shape=jax.ShapeDtypeStruct((n_pad, D_MODEL), jnp.float32),
        interpret=_INTERPRET,
    )(off8, cnts, W_K, W_V, kmax, vmax, a_sorted)


def _ffn1_kernel(comb_ref, win_ref, h_ref, hmax_ref):
    cmax = jnp.max(jnp.abs(win_ref[...]), axis=0, keepdims=True)
    cs = 127.0 / jnp.maximum(cmax, 1e-8)
    wq = jnp.round(win_ref[...] * cs) / cs
    h = jnp.dot(comb_ref[...], wq, preferred_element_type=jnp.float32)
    h_ref[...] = h

    @pl.when(pl.program_id(0) == 0)
    def _init():
        hmax_ref[...] = jnp.zeros_like(hmax_ref)
    hmax_ref[...] = jnp.maximum(hmax_ref[...],
                                jnp.max(jnp.abs(h)).reshape(1, 1))


def _ffn1(comb, W_in, tile_rows):
    grid = comb.shape[0] // tile_rows
    return pl.pallas_call(
        _ffn1_kernel,
        grid=(grid,),
        in_specs=[
            pl.BlockSpec((tile_rows, D_MODEL), lambda i: (i, 0)),
            pl.BlockSpec((D_MODEL, D_FF), lambda i: (0, 0)),
        ],
        out_specs=[
            pl.BlockSpec((tile_rows, D_FF), lambda i: (i, 0)),
            pl.BlockSpec((1, 1), lambda i: (0, 0)),
        ],
        out_shape=[
            jax.ShapeDtypeStruct((comb.shape[0], D_FF), jnp.float32),
            jax.ShapeDtypeStruct((1, 1), jnp.float32),
        ],
        interpret=_INTERPRET,
    )(comb, W_in)


def _ffn2_kernel(h_ref, hmax_ref, xsim_ref, wout_ref, wskip_ref,
                 g_ref, b_ref, o_ref):
    hs = 32767.0 / jnp.maximum(hmax_ref[...], 1e-8)
    h = jnp.round(h_ref[...] * hs) / hs
    m = jnp.mean(h, axis=-1, keepdims=True)
    v = jnp.mean((h - m) ** 2, axis=-1, keepdims=True)
    ln = (h - m) * lax.rsqrt(v + 1e-5) * g_ref[...] + b_ref[...]

    rmax = jnp.max(jnp.abs(wout_ref[...]), axis=1, keepdims=True)
    rs = 127.0 / jnp.maximum(rmax, 1e-8)
    woq = jnp.round(wout_ref[...] * rs) / rs

    smax = jnp.max(jnp.abs(wskip_ref[...]))
    ss = 127.0 / jnp.maximum(smax, 1e-8)
    wsq = jnp.round(wskip_ref[...] * ss) / ss

    o_ref[...] = (
        jnp.dot(ln, woq, preferred_element_type=jnp.float32)
        + jnp.dot(xsim_ref[...], wsq, preferred_element_type=jnp.float32)
    )


def _ffn2(hidden, hmax, x_sim, W_out, W_skip, ln2_g, ln2_b, tile_rows):
    grid = hidden.shape[0] // tile_rows
    return pl.pallas_call(
        _ffn2_kernel,
        grid=(grid,),
        in_specs=[
            pl.BlockSpec((tile_rows, D_FF), lambda i: (i, 0)),
            pl.BlockSpec((1, 1), lambda i: (0, 0)),
            pl.BlockSpec((tile_rows, D_MODEL), lambda i: (i, 0)),
            pl.BlockSpec((D_FF, D_MODEL), lambda i: (0, 0)),
            pl.BlockSpec((D_MODEL, D_MODEL), lambda i: (0, 0)),
            pl.BlockSpec((1, D_FF), lambda i: (0, 0)),
            pl.BlockSpec((1, D_FF), lambda i: (0, 0)),
        ],
        out_specs=pl.BlockSpec((tile_rows, D_MODEL), lambda i: (i, 0)),
        out_shape=jax.ShapeDtypeStruct((hidden.shape[0], D_MODEL), jnp.float32),
        interpret=_INTERPRET,
    )(hidden, hmax, x_sim, W_out, W_skip,
      ln2_g.reshape(1, -1), ln2_b.reshape(1, -1))


def kernel(x, input_ids, W_K, W_V, W_in, W_out, W_skip,
           ln1_g, ln1_b, ln2_g, ln2_b):
    B, T, D = x.shape
    n_tok = B * T
    n_pad = 4096
    assert n_tok + N_BUCKET * 7 + TOK_CHUNK <= n_pad
    x2d = x.reshape(n_tok, D)

    ids = input_ids.reshape(n_tok).astype(jnp.int32) % N_BUCKET
    onehot = (ids[:, None] == jnp.arange(N_BUCKET, dtype=jnp.int32)[None, :])
    onehot = onehot.astype(jnp.int32)
    occ = jnp.cumsum(onehot, axis=0)
    counts = occ[-1].astype(jnp.int32)
    rank = jnp.take_along_axis(occ, ids[:, None], axis=1)[:, 0] - 1
    aligned = ((counts + 7) // 8) * 8
    aligned_off = jnp.concatenate([jnp.zeros((1,), jnp.int32),
                                   jnp.cumsum(aligned).astype(jnp.int32)])
    off8 = (aligned_off // 8).astype(jnp.int32)
    inv_pos = (aligned_off[ids] + rank).astype(jnp.int32)
    slot_token = jnp.zeros((n_pad,), jnp.int32).at[inv_pos].set(
        jnp.arange(n_tok, dtype=jnp.int32))

    xmax = _global_maxabs(x2d, 256)
    kmax, vmax = _global_maxabs2(W_K, W_V, 2048)

    x_sim, a_nat = _quantize_x_ln(x2d, xmax, ln1_g, ln1_b, 256)
    a_sorted = _tc_gather_rows(a_nat, slot_token, n_pad)

    comb_sorted = _bucket_attention(W_K, W_V, kmax, vmax, a_sorted,
                                    off8, counts, n_pad)
    comb = _tc_gather_rows(comb_sorted, inv_pos, n_tok)

    hidden, hmax = _ffn1(comb, W_in, 256)
    out = _ffn2(hidden, hmax, x_sim, W_out, W_skip, ln2_g, ln2_b, 256)
    return out.reshape(B, T, D)
